# baseline (speedup 1.0000x reference)
import jax
import jax.numpy as jnp
from jax import lax
from jax.experimental import pallas as pl
from jax.experimental.pallas import tpu as pltpu

B, N, F = 8, 1024, 128
C = 32
DIM = 4 * C
K = 64
C1_OUT = 16
C2_OUT = 32
C2_W = 5
DENSE = 128
NUM_CLASSES = 2
OUT_W = 128
GPS = 2


def _dot(a, b):
    return jnp.dot(a, b, preferred_element_type=jnp.float32)


def _dot_rt(a, b):
    return lax.dot_general(a, b, (((1,), (1,)), ((), ())),
                           preferred_element_type=jnp.float32)


def _body(ns_ref, a_ref, xt_ref, w1t_ref, w2t_ref, w3t_ref, w4t_ref,
          c1kt_ref, c1b_ref, c2kt_ref, c2b_ref, dwt_ref, db_ref,
          owt_ref, ob_ref, out_ref, cat_ref, v_ref):
    step = pl.program_id(0)

    for g in range(GPS):
        A = a_ref[g]
        XT = xt_ref[g]

        Z1 = jnp.tanh(_dot_rt(_dot(w1t_ref[...], XT), A))
        Z2 = jnp.tanh(_dot_rt(_dot(w2t_ref[...], Z1), A))
        Z3 = jnp.tanh(_dot_rt(_dot(w3t_ref[...], Z2), A))
        Z4 = jnp.tanh(_dot_rt(_dot(w4t_ref[...], Z3), A))
        cat_ref[step * GPS + g] = jnp.concatenate(
            [Z1, Z2, Z3, Z4], axis=0)
        v_ref[step * GPS + g] = Z4[C - 1:C, :]

    @pl.when(step == B // GPS - 1)
    def _tail():
        v_all = jnp.concatenate([v_ref[g] for g in range(B)], axis=0)
        lane = lax.broadcasted_iota(jnp.int32, (B, N), 1)
        kcol = lax.broadcasted_iota(jnp.int32, (B, K), 1)

        def sel(k, carry):
            vv, idxs = carry
            m = jnp.max(vv, axis=1, keepdims=True)
            idx = jnp.min(jnp.where(vv == m, lane, N),
                          axis=1, keepdims=True)
            idxs = jnp.where(kcol == k, idx, idxs)
            vv = jnp.where(lane == idx, -2.0, vv)
            return vv, idxs

        _, idxs = lax.fori_loop(
            0, K, sel, (v_all, jnp.zeros((B, K), jnp.int32)))

        rn = lax.broadcasted_iota(jnp.int32, (N, K), 0)
        kc2 = lax.broadcasted_iota(jnp.int32, (N, K), 1)
        pieces = []
        for g in range(B):
            Sg = jnp.where((rn == idxs[g:g + 1, :]) & (kc2 < ns_ref[g]),
                           1.0, 0.0)
            pieces.append(_dot(cat_ref[g], Sg))
        topk = jnp.concatenate(pieces, axis=1)

        h1 = jax.nn.relu(_dot(c1kt_ref[...], topk) + c1b_ref[...])

        pr = lax.broadcasted_iota(jnp.int32, (B * K, B * K // 2), 0)
        pc = lax.broadcasted_iota(jnp.int32, (B * K, B * K // 2), 1)
        Ev = jnp.where(pr == 2 * pc, 1.0, 0.0)
        Od = jnp.where(pr == 2 * pc + 1, 1.0, 0.0)
        pooled = jnp.maximum(_dot(h1, Ev), _dot(h1, Od))

        P = K // 2
        sq = lax.broadcasted_iota(jnp.int32, (B * P, B * P), 0)
        tq = lax.broadcasted_iota(jnp.int32, (B * P, B * P), 1)
        same = (sq >> 5) == (tq >> 5)
        h2 = c2b_ref[...]
        for d in range(C2_W):
            Qd = jnp.where((sq == tq + (d - C2_W // 2)) & same, 1.0, 0.0)
            h2 = h2 + _dot(c2kt_ref[:, d * C1_OUT:(d + 1) * C1_OUT],
                           _dot(pooled, Qd))
        h2 = jax.nn.relu(h2)

        rr = lax.broadcasted_iota(jnp.int32, (B * P, B), 0)
        gg = lax.broadcasted_iota(jnp.int32, (B * P, B), 1)
        d1 = db_ref[...]
        for t in range(P):
            St = jnp.where(rr == (gg << 5) + t, 1.0, 0.0)
            d1 = d1 + _dot(dwt_ref[:, t * C2_OUT:(t + 1) * C2_OUT],
                           _dot(h2, St))
        d1 = jax.nn.relu(d1)

        logits = _dot(owt_ref[...], d1) + ob_ref[...]
        l0 = logits[0:1, :]
        l1 = logits[1:2, :]
        m = jnp.maximum(l0, l1)
        e0 = jnp.exp(l0 - m)
        e1 = jnp.exp(l1 - m)
        tot = e0 + e1
        predf = jnp.where(l1 > l0, 1.0, 0.0)

        stack = jnp.concatenate(
            [l0, l1, e0 / tot, e1 / tot, predf,
             jnp.zeros((3, B), jnp.float32)], axis=0)
        i8a = lax.broadcasted_iota(jnp.int32, (8, 8), 0)
        i8b = lax.broadcasted_iota(jnp.int32, (8, 8), 1)
        eye8 = jnp.where(i8a == i8b, 1.0, 0.0)
        tr = _dot_rt(eye8, stack)
        zpad = jnp.zeros((B, OUT_W - 8), jnp.float32)
        out_ref[...] = jnp.concatenate([tr, zpad], axis=1)


def kernel(D_inverse, A_tilde, X, nodes_size_list, is_train, W1, W2, W3, W4,
           conv1_k, conv1_b, conv2_k, conv2_b, dense_W, dense_b, out_W, out_b):
    del D_inverse, is_train

    XT = X.transpose(0, 2, 1)
    c1kt = conv1_k.reshape(DIM, C1_OUT).T
    c2kt = conv2_k.reshape(C2_W * C1_OUT, C2_OUT).T

    grid_spec = pltpu.PrefetchScalarGridSpec(
        num_scalar_prefetch=1,
        grid=(B // GPS,),
        in_specs=[
            pl.BlockSpec((GPS, N, N), lambda b, ns: (b, 0, 0)),
            pl.BlockSpec((GPS, F, N), lambda b, ns: (b, 0, 0)),
            pl.BlockSpec((C, F), lambda b, ns: (0, 0)),
            pl.BlockSpec((C, C), lambda b, ns: (0, 0)),
            pl.BlockSpec((C, C), lambda b, ns: (0, 0)),
            pl.BlockSpec((C, C), lambda b, ns: (0, 0)),
            pl.BlockSpec((C1_OUT, DIM), lambda b, ns: (0, 0)),
            pl.BlockSpec((C1_OUT, 1), lambda b, ns: (0, 0)),
            pl.BlockSpec((C2_OUT, C2_W * C1_OUT), lambda b, ns: (0, 0)),
            pl.BlockSpec((C2_OUT, 1), lambda b, ns: (0, 0)),
            pl.BlockSpec((DENSE, (K // 2) * C2_OUT), lambda b, ns: (0, 0)),
            pl.BlockSpec((DENSE, 1), lambda b, ns: (0, 0)),
            pl.BlockSpec((NUM_CLASSES, DENSE), lambda b, ns: (0, 0)),
            pl.BlockSpec((NUM_CLASSES, 1), lambda b, ns: (0, 0)),
        ],
        out_specs=pl.BlockSpec((B, OUT_W), lambda b, ns: (0, 0)),
        scratch_shapes=[
            pltpu.VMEM((B, DIM, N), jnp.float32),
            pltpu.VMEM((B, 1, N), jnp.float32),
        ],
    )

    out = pl.pallas_call(
        _body,
        grid_spec=grid_spec,
        out_shape=jax.ShapeDtypeStruct((B, OUT_W), jnp.float32),
    )(nodes_size_list.astype(jnp.int32), A_tilde, XT,
      W1.T, W2.T, W3.T, W4.T,
      c1kt, conv1_b.reshape(C1_OUT, 1), c2kt, conv2_b.reshape(C2_OUT, 1),
      dense_W.T, dense_b.reshape(DENSE, 1),
      out_W.T, out_b.reshape(NUM_CLASSES, 1))

    logits = out[:, 0:2]
    pos_score = out[:, 2:4]
    pred = out[:, 4].astype(jnp.int32)
    return (pos_score, logits, pred)


# speedup vs baseline: 1.0512x; 1.0512x over previous
import jax
import jax.numpy as jnp
from jax import lax
from jax.experimental import pallas as pl
from jax.experimental.pallas import tpu as pltpu

B, N, F = 8, 1024, 128
C = 32
DIM = 4 * C
K = 64
C1_OUT = 16
C2_OUT = 32
C2_W = 5
DENSE = 128
NUM_CLASSES = 2
OUT_W = 128


def _dot(a, b):
    return jnp.dot(a, b, preferred_element_type=jnp.float32,
                   precision=lax.Precision.DEFAULT)


GPS = 4


def _body(ns_ref, a_ref, x_ref, w1_ref, w2_ref, w3_ref, w4_ref,
          c1k_ref, c1b_ref, c2k_ref, c2b_ref, dw_ref, db_ref,
          ow_ref, ob_ref, out_ref, cat_ref, v_ref):
    step = pl.program_id(0)

    for g in range(GPS):
        A = a_ref[g]
        Xb = x_ref[g]

        Z1 = jnp.tanh(_dot(A, _dot(Xb, w1_ref[...])))
        Z2 = jnp.tanh(_dot(A, _dot(Z1, w2_ref[...])))
        Z3 = jnp.tanh(_dot(A, _dot(Z2, w3_ref[...])))
        Z4 = jnp.tanh(_dot(A, _dot(Z3, w4_ref[...])))
        cat_ref[step * GPS + g] = jnp.concatenate(
            [Z1, Z2, Z3, Z4], axis=1)

        v_ref[step * GPS + g] = jnp.transpose(Z4[:, C - 1:C])

    @pl.when(step == B // GPS - 1)
    def _tail():
        v_all = jnp.concatenate([v_ref[g] for g in range(B)], axis=0)
        lane = lax.broadcasted_iota(jnp.int32, (B, N), 1)
        kcol = lax.broadcasted_iota(jnp.int32, (B, K), 1)

        vv = v_all
        idxs = jnp.zeros((B, K), jnp.float32)
        for k in range(K):
            m = jnp.max(vv, axis=1, keepdims=True)
            idx = jnp.min(jnp.where(vv == m, lane, N),
                          axis=1, keepdims=True)
            idxs = jnp.where(kcol == k, idx.astype(jnp.float32), idxs)
            vv = jnp.where(lane == idx, -2.0, vv)
        idxs_t = jnp.transpose(idxs).astype(jnp.int32)

        ki = lax.broadcasted_iota(jnp.int32, (K, 1), 0)
        coln = lax.broadcasted_iota(jnp.int32, (K, N), 1)
        pieces = []
        for g in range(B):
            Sg = jnp.where((coln == idxs_t[:, g:g + 1]) & (ki < ns_ref[g]),
                           1.0, 0.0)
            pieces.append(_dot(Sg, cat_ref[g]))
        topk = jnp.concatenate(pieces, axis=0)

        h1 = jax.nn.relu(_dot(topk, c1k_ref[...]) + c1b_ref[...])

        pr = lax.broadcasted_iota(jnp.int32, (B * K // 2, B * K), 0)
        pc = lax.broadcasted_iota(jnp.int32, (B * K // 2, B * K), 1)
        Ev = jnp.where(pc == 2 * pr, 1.0, 0.0)
        Od = jnp.where(pc == 2 * pr + 1, 1.0, 0.0)
        pooled = jnp.maximum(_dot(Ev, h1), _dot(Od, h1))

        P = K // 2
        sr = lax.broadcasted_iota(jnp.int32, (B * P, B * P), 0)
        sc = lax.broadcasted_iota(jnp.int32, (B * P, B * P), 1)
        same = (sr >> 5) == (sc >> 5)
        h2 = c2b_ref[...]
        for d in range(C2_W):
            Pd = jnp.where((sc == sr + (d - C2_W // 2)) & same, 1.0, 0.0)
            h2 = h2 + _dot(_dot(Pd, pooled),
                           c2k_ref[d * C1_OUT:(d + 1) * C1_OUT, :])
        h2 = jax.nn.relu(h2)

        gr = lax.broadcasted_iota(jnp.int32, (B, B * P), 0)
        gc = lax.broadcasted_iota(jnp.int32, (B, B * P), 1)
        d1 = db_ref[...]
        for t in range(P):
            St = jnp.where(gc == (gr << 5) + t, 1.0, 0.0)
            d1 = d1 + _dot(_dot(St, h2),
                           dw_ref[t * C2_OUT:(t + 1) * C2_OUT, :])
        d1 = jax.nn.relu(d1)

        logits = _dot(d1, ow_ref[...]) + ob_ref[...]
        l0 = logits[:, 0:1]
        l1 = logits[:, 1:2]
        m = jnp.maximum(l0, l1)
        e0 = jnp.exp(l0 - m)
        e1 = jnp.exp(l1 - m)
        tot = e0 + e1
        predf = jnp.where(l1 > l0, 1.0, 0.0)

        out_lane = lax.broadcasted_iota(jnp.int32, (B, OUT_W), 1)
        out_ref[...] = jnp.where(out_lane == 0, l0,
                       jnp.where(out_lane == 1, l1,
                       jnp.where(out_lane == 2, e0 / tot,
                       jnp.where(out_lane == 3, e1 / tot,
                       jnp.where(out_lane == 4, predf, 0.0)))))


def kernel(D_inverse, A_tilde, X, nodes_size_list, is_train, W1, W2, W3, W4,
           conv1_k, conv1_b, conv2_k, conv2_b, dense_W, dense_b, out_W, out_b):
    del D_inverse, is_train

    c1k = conv1_k.reshape(DIM, C1_OUT)
    c2k = conv2_k.reshape(C2_W * C1_OUT, C2_OUT)

    grid_spec = pltpu.PrefetchScalarGridSpec(
        num_scalar_prefetch=1,
        grid=(B // GPS,),
        in_specs=[
            pl.BlockSpec((GPS, N, N), lambda b, ns: (b, 0, 0)),
            pl.BlockSpec((GPS, N, F), lambda b, ns: (b, 0, 0)),
            pl.BlockSpec((F, C), lambda b, ns: (0, 0)),
            pl.BlockSpec((C, C), lambda b, ns: (0, 0)),
            pl.BlockSpec((C, C), lambda b, ns: (0, 0)),
            pl.BlockSpec((C, C), lambda b, ns: (0, 0)),
            pl.BlockSpec((DIM, C1_OUT), lambda b, ns: (0, 0)),
            pl.BlockSpec((1, C1_OUT), lambda b, ns: (0, 0)),
            pl.BlockSpec((C2_W * C1_OUT, C2_OUT), lambda b, ns: (0, 0)),
            pl.BlockSpec((1, C2_OUT), lambda b, ns: (0, 0)),
            pl.BlockSpec(((K // 2) * C2_OUT, DENSE), lambda b, ns: (0, 0)),
            pl.BlockSpec((1, DENSE), lambda b, ns: (0, 0)),
            pl.BlockSpec((DENSE, NUM_CLASSES), lambda b, ns: (0, 0)),
            pl.BlockSpec((1, NUM_CLASSES), lambda b, ns: (0, 0)),
        ],
        out_specs=pl.BlockSpec((B, OUT_W), lambda b, ns: (0, 0)),
        scratch_shapes=[
            pltpu.VMEM((B, N, DIM), jnp.float32),
            pltpu.VMEM((B, 1, N), jnp.float32),
        ],
    )

    out = pl.pallas_call(
        _body,
        grid_spec=grid_spec,
        out_shape=jax.ShapeDtypeStruct((B, OUT_W), jnp.float32),
    )(nodes_size_list.astype(jnp.int32), A_tilde, X, W1, W2, W3, W4,
      c1k, conv1_b.reshape(1, C1_OUT), c2k, conv2_b.reshape(1, C2_OUT),
      dense_W, dense_b.reshape(1, DENSE), out_W, out_b.reshape(1, NUM_CLASSES))

    logits = out[:, 0:2]
    pos_score = out[:, 2:4]
    pred = out[:, 4].astype(jnp.int32)
    return (pos_score, logits, pred)


# speedup vs baseline: 1.0905x; 1.0374x over previous
import jax
import jax.numpy as jnp
from jax import lax
from jax.experimental import pallas as pl
from jax.experimental.pallas import tpu as pltpu

B, N, F = 8, 1024, 128
C = 32
DIM = 4 * C
K = 64
C1_OUT = 16
C2_OUT = 32
C2_W = 5
DENSE = 128
NUM_CLASSES = 2
OUT_W = 128


def _dot(a, b):
    return jnp.dot(a, b, preferred_element_type=jnp.float32,
                   precision=lax.Precision.DEFAULT)


GPS = 4


def _body(ns_ref, a_ref, x_ref, w1_ref, w2_ref, w3_ref, w4_ref,
          c1k_ref, c1b_ref, c2k_ref, c2b_ref, dw_ref, db_ref,
          ow_ref, ob_ref, out_ref, cat_ref, rank_ref):
    step = pl.program_id(0)

    for g in range(GPS):
        A = a_ref[g]
        Xb = x_ref[g]

        Z1 = jnp.tanh(_dot(A, _dot(Xb, w1_ref[...])))
        Z2 = jnp.tanh(_dot(A, _dot(Z1, w2_ref[...])))
        Z3 = jnp.tanh(_dot(A, _dot(Z2, w3_ref[...])))
        Z4 = jnp.tanh(_dot(A, _dot(Z3, w4_ref[...])))
        cat_ref[step * GPS + g] = jnp.concatenate(
            [Z1, Z2, Z3, Z4], axis=1)

        v_col = Z4[:, C - 1:C]
        v_row = jnp.transpose(v_col)
        n_i = lax.broadcasted_iota(jnp.int32, (N, N), 0)
        j_i = lax.broadcasted_iota(jnp.int32, (N, N), 1)
        G = jnp.where((v_col > v_row) | ((v_col == v_row) & (n_i < j_i)),
                      1.0, 0.0)
        rank_ref[step * GPS + g] = _dot(
            jnp.ones((1, N), jnp.float32), G)

    @pl.when(step == B // GPS - 1)
    def _tail():
        ki = lax.broadcasted_iota(jnp.int32, (K, 1), 0)
        pieces = []
        for g in range(B):
            rk = rank_ref[g].astype(jnp.int32)
            Sg = jnp.where((rk == ki) & (ki < ns_ref[g]),
                           1.0, 0.0)
            pieces.append(_dot(Sg, cat_ref[g]))
        topk = jnp.concatenate(pieces, axis=0)

        h1 = jax.nn.relu(_dot(topk, c1k_ref[...]) + c1b_ref[...])

        pr = lax.broadcasted_iota(jnp.int32, (B * K // 2, B * K), 0)
        pc = lax.broadcasted_iota(jnp.int32, (B * K // 2, B * K), 1)
        Ev = jnp.where(pc == 2 * pr, 1.0, 0.0)
        Od = jnp.where(pc == 2 * pr + 1, 1.0, 0.0)
        pooled = jnp.maximum(_dot(Ev, h1), _dot(Od, h1))

        P = K // 2
        sr = lax.broadcasted_iota(jnp.int32, (B * P, B * P), 0)
        sc = lax.broadcasted_iota(jnp.int32, (B * P, B * P), 1)
        same = (sr >> 5) == (sc >> 5)
        h2 = c2b_ref[...]
        for d in range(C2_W):
            Pd = jnp.where((sc == sr + (d - C2_W // 2)) & same, 1.0, 0.0)
            h2 = h2 + _dot(_dot(Pd, pooled),
                           c2k_ref[d * C1_OUT:(d + 1) * C1_OUT, :])
        h2 = jax.nn.relu(h2)

        gr = lax.broadcasted_iota(jnp.int32, (B, B * P), 0)
        gc = lax.broadcasted_iota(jnp.int32, (B, B * P), 1)
        d1 = db_ref[...]
        for t in range(P):
            St = jnp.where(gc == (gr << 5) + t, 1.0, 0.0)
            d1 = d1 + _dot(_dot(St, h2),
                           dw_ref[t * C2_OUT:(t + 1) * C2_OUT, :])
        d1 = jax.nn.relu(d1)

        logits = _dot(d1, ow_ref[...]) + ob_ref[...]
        l0 = logits[:, 0:1]
        l1 = logits[:, 1:2]
        m = jnp.maximum(l0, l1)
        e0 = jnp.exp(l0 - m)
        e1 = jnp.exp(l1 - m)
        tot = e0 + e1
        predf = jnp.where(l1 > l0, 1.0, 0.0)

        out_lane = lax.broadcasted_iota(jnp.int32, (B, OUT_W), 1)
        out_ref[...] = jnp.where(out_lane == 0, l0,
                       jnp.where(out_lane == 1, l1,
                       jnp.where(out_lane == 2, e0 / tot,
                       jnp.where(out_lane == 3, e1 / tot,
                       jnp.where(out_lane == 4, predf, 0.0)))))


def kernel(D_inverse, A_tilde, X, nodes_size_list, is_train, W1, W2, W3, W4,
           conv1_k, conv1_b, conv2_k, conv2_b, dense_W, dense_b, out_W, out_b):
    del D_inverse, is_train

    c1k = conv1_k.reshape(DIM, C1_OUT)
    c2k = conv2_k.reshape(C2_W * C1_OUT, C2_OUT)

    grid_spec = pltpu.PrefetchScalarGridSpec(
        num_scalar_prefetch=1,
        grid=(B // GPS,),
        in_specs=[
            pl.BlockSpec((GPS, N, N), lambda b, ns: (b, 0, 0)),
            pl.BlockSpec((GPS, N, F), lambda b, ns: (b, 0, 0)),
            pl.BlockSpec((F, C), lambda b, ns: (0, 0)),
            pl.BlockSpec((C, C), lambda b, ns: (0, 0)),
            pl.BlockSpec((C, C), lambda b, ns: (0, 0)),
            pl.BlockSpec((C, C), lambda b, ns: (0, 0)),
            pl.BlockSpec((DIM, C1_OUT), lambda b, ns: (0, 0)),
            pl.BlockSpec((1, C1_OUT), lambda b, ns: (0, 0)),
            pl.BlockSpec((C2_W * C1_OUT, C2_OUT), lambda b, ns: (0, 0)),
            pl.BlockSpec((1, C2_OUT), lambda b, ns: (0, 0)),
            pl.BlockSpec(((K // 2) * C2_OUT, DENSE), lambda b, ns: (0, 0)),
            pl.BlockSpec((1, DENSE), lambda b, ns: (0, 0)),
            pl.BlockSpec((DENSE, NUM_CLASSES), lambda b, ns: (0, 0)),
            pl.BlockSpec((1, NUM_CLASSES), lambda b, ns: (0, 0)),
        ],
        out_specs=pl.BlockSpec((B, OUT_W), lambda b, ns: (0, 0)),
        scratch_shapes=[
            pltpu.VMEM((B, N, DIM), jnp.float32),
            pltpu.VMEM((B, 1, N), jnp.float32),
        ],
    )

    out = pl.pallas_call(
        _body,
        grid_spec=grid_spec,
        out_shape=jax.ShapeDtypeStruct((B, OUT_W), jnp.float32),
    )(nodes_size_list.astype(jnp.int32), A_tilde, X, W1, W2, W3, W4,
      c1k, conv1_b.reshape(1, C1_OUT), c2k, conv2_b.reshape(1, C2_OUT),
      dense_W, dense_b.reshape(1, DENSE), out_W, out_b.reshape(1, NUM_CLASSES))

    logits = out[:, 0:2]
    pos_score = out[:, 2:4]
    pred = out[:, 4].astype(jnp.int32)
    return (pos_score, logits, pred)
